# Initial kernel scaffold; baseline (speedup 1.0000x reference)
import jax
import jax.numpy as jnp
from jax.experimental import pallas as pl


def kernel(x, edge_index, src, dst, W1a, b1a, W1b, b1b, W2a, b2a, W2b, b2b):
    raise NotImplementedError("write your pallas kernel here")


# speedup vs baseline: 49.8244x; 49.8244x over previous
import functools

import jax
import jax.numpy as jnp
from jax import lax
from jax.experimental import pallas as pl
from jax.experimental.pallas import tpu as pltpu
from jax.experimental.pallas import tpu_sc as plsc

_L = 16
_D = 128


def _sc_hist_body(NP, SH, CH, NS, E, edge_ref, sd_ref, out_ref,
                  sd_v, srcc_v, dstc_v, deg_v, cs_v, cd_v):
    c = lax.axis_index("c")
    sid = lax.axis_index("s")
    wid = c * NS + sid
    z16 = jnp.zeros((_L,), jnp.float32)

    def zero_body(i, _):
        deg_v[pl.ds(i * _L, _L)] = z16
        cs_v[pl.ds(i * _L, _L)] = z16
        cd_v[pl.ds(i * _L, _L)] = z16
        return 0
    lax.fori_loop(0, NP // _L, zero_body, 0)

    pltpu.sync_copy(sd_ref, sd_v)
    s_splat = sd_v[pl.ds(0, _L)]
    d_splat = sd_v[pl.ds(_L, _L)]
    ones16 = jnp.ones((_L,), jnp.float32)

    for ch in range(SH // CH):
        base = wid * SH + ch * CH
        pltpu.sync_copy(edge_ref.at[pl.ds(base, CH)], srcc_v)
        pltpu.sync_copy(edge_ref.at[pl.ds(E + base, CH)], dstc_v)

        def body(i, _):
            sv = srcc_v[pl.ds(i * _L, _L)]
            dv = dstc_v[pl.ds(i * _L, _L)]
            plsc.addupdate_scatter(deg_v, [dv], ones16, mask=dv >= 0)
            plsc.addupdate_scatter(cs_v, [sv], ones16, mask=dv == s_splat)
            plsc.addupdate_scatter(cd_v, [sv], ones16, mask=dv == d_splat)
            return 0
        lax.fori_loop(0, CH // _L, body, 0)

    pltpu.sync_copy(deg_v, out_ref.at[pl.ds((wid * 3 + 0) * NP, NP)])
    pltpu.sync_copy(cs_v, out_ref.at[pl.ds((wid * 3 + 1) * NP, NP)])
    pltpu.sync_copy(cd_v, out_ref.at[pl.ds((wid * 3 + 2) * NP, NP)])


def _sc_msg_body(NP, E, RBP, CH, NS, edge_ref, mdis_ref, g_ref, zz_ref,
                 out_ref, mdis_v, srcc_v, dstc_v, packed, src_stage, dst_stage,
                 rowbuf, sem, acc):
    c = lax.axis_index("c")
    sid = lax.axis_index("s")
    SH = E // NS
    ACCR = NP + _D
    G = 64
    iota16 = lax.iota(jnp.int32, _L)

    pltpu.sync_copy(mdis_ref, mdis_v)

    head_base = c * NP
    fill = jnp.full((_L,), NP * 16384, jnp.int32)

    def prefill(i, _):
        p = i * _L + iota16
        ri = jnp.right_shift(p, 7)
        ci = jnp.bitwise_and(p, _D - 1)
        plsc.store_scatter(packed, [ri, ci], fill)
        return 0
    lax.fori_loop(0, RBP * _D // _L, prefill, 0)

    rows_pt = ACCR // NS
    done = 0
    for chunk in [_D] * (rows_pt // _D) + ([rows_pt % _D] if rows_pt % _D else []):
        pltpu.sync_copy(zz_ref.at[pl.ds(0, chunk)],
                        acc.at[pl.ds(sid * rows_pt + done, chunk)])
        done += chunk
    plsc.subcore_barrier()

    off = jnp.int32(0)
    for ch in range(SH // CH):
        base = sid * SH + ch * CH
        pltpu.sync_copy(edge_ref.at[pl.ds(base, CH)], srcc_v)
        pltpu.sync_copy(edge_ref.at[pl.ds(E + base, CH)], dstc_v)

        def body(i, off):
            sv = srcc_v[pl.ds(i * _L, _L)]
            dv = dstc_v[pl.ds(i * _L, _L)]
            mk = plsc.load_gather(mdis_v, [dv])
            m = mk > 0.0
            mi = m.astype(jnp.int32)
            pos = off + plsc.cumsum(mi) - mi
            ri = jnp.right_shift(pos, 7)
            ci = jnp.bitwise_and(pos, _D - 1)
            plsc.store_scatter(packed, [ri, ci], dv * 16384 + sv, mask=m)
            return off + jnp.sum(mi)
        off = lax.fori_loop(0, CH // _L, body, off)

    zero16 = jnp.zeros((_L,), jnp.int32)
    nch = jnp.right_shift(off + (G - 1), 6)

    def chunk_body(i, _):
        i_splat = jnp.full((_L,), jnp.right_shift(i, 1), jnp.int32)
        lane0 = jnp.bitwise_and(i, 1) * G

        def upk(j, _):
            lanes = lane0 + j * _L + iota16
            pv = plsc.load_gather(packed, [i_splat, lanes])
            dv = jnp.right_shift(pv, 14)
            sv = jnp.bitwise_and(pv, 16383)
            src_stage[pl.ds(j * _L, _L)] = sv + head_base
            plsc.store_scatter(dst_stage, [zero16, j * _L + iota16], dv)
            return 0
        lax.fori_loop(0, G // _L, upk, 0)
        pltpu.async_copy(g_ref.at[src_stage], rowbuf, sem).wait()

        def grp(j, _):
            rows16 = j * _L + iota16
            dvec = plsc.load_gather(dst_stage, [zero16, rows16])
            w16 = plsc.load_gather(mdis_v, [dvec])

            def cbody(col, _):
                colsp = jnp.full((_L,), col, jnp.int32)
                cvec = plsc.load_gather(rowbuf, [rows16, colsp])
                plsc.store_scatter(rowbuf, [rows16, colsp], cvec * w16)
                return 0
            lax.fori_loop(0, _D, cbody, 0)
            return 0
        lax.fori_loop(0, G // _L, grp, 0)
        pltpu.sync_copy(rowbuf, acc.at[dst_stage.at[0]], add=True)
        return 0
    lax.fori_loop(0, nch, chunk_body, 0)

    plsc.subcore_barrier()

    rows_wb = NP // NS
    done = 0
    for _unused in range(rows_wb // _D):
        pltpu.sync_copy(acc.at[pl.ds(sid * rows_wb + done, _D)],
                        out_ref.at[c, pl.ds(sid * rows_wb + done, _D)])
        done += _D


def _tc_prep_body(x_ref, hist_ref, w1_ref, w2_ref, sd_ref,
                  g_ref, dis_ref, mark_ref, ws_ref, wd_ref):
    pid = pl.program_id(0)
    h = jnp.sum(hist_ref[...], axis=0)
    deg = h[0:1, :] + 1.0
    cs = h[1:2, :]
    cd = h[2:3, :]
    dis = 1.0 / jnp.sqrt(deg)
    ids = pid * _D + lax.broadcasted_iota(jnp.int32, (1, _D), 1)
    s = sd_ref[0, 0]
    d = sd_ref[0, 1]
    marked = (cs + cd > 0.0) | (ids == s) | (ids == d)
    dis_ref[0] = dis
    mark_ref[0] = jnp.where(marked, dis, -dis)
    ws_ref[0] = cs * dis
    wd_ref[0] = cd * dis
    ey = (lax.broadcasted_iota(jnp.int32, (_D, _D), 0)
          == lax.broadcasted_iota(jnp.int32, (_D, _D), 1)).astype(jnp.float32)
    dis_col = lax.dot_general(ey, dis, (((1,), (1,)), ((), ())),
                              preferred_element_type=jnp.float32)
    xb = x_ref[...]
    g_ref[0] = jnp.dot(xb, w1_ref[...], preferred_element_type=jnp.float32) * dis_col
    g_ref[1] = jnp.dot(xb, w2_ref[...], preferred_element_type=jnp.float32) * dis_col


def _tc_epi_body(acc_ref, g_ref, dis_ref, ws_ref, wd_ref, b1a_ref, b2a_ref,
                 w1b_ref, w2b_ref, sd_ref, out_ref, yscr, t1scr, t2scr):
    pid = pl.program_id(0)
    nb = pl.num_programs(0)

    @pl.when(pid == 0)
    def _():
        yscr[...] = jnp.zeros((8, _D), jnp.float32)

    dis = dis_ref[0]
    ey = (lax.broadcasted_iota(jnp.int32, (_D, _D), 0)
          == lax.broadcasted_iota(jnp.int32, (_D, _D), 1)).astype(jnp.float32)
    dis_col = lax.dot_general(ey, dis, (((1,), (1,)), ((), ())),
                              preferred_element_type=jnp.float32)
    z1 = acc_ref[0] + g_ref[0] * dis_col + b1a_ref[...]
    z2 = acc_ref[1] + g_ref[1] * dis_col + b2a_ref[...]
    t1 = jnp.dot(jax.nn.relu(z1), w1b_ref[...], preferred_element_type=jnp.float32)
    t2 = jnp.dot(jax.nn.relu(z2), w2b_ref[...], preferred_element_type=jnp.float32)
    ws = ws_ref[0]
    wd = wd_ref[0]
    y = jnp.concatenate(
        [jnp.dot(ws, t1, preferred_element_type=jnp.float32),
         jnp.dot(wd, t1, preferred_element_type=jnp.float32),
         jnp.dot(ws, t2, preferred_element_type=jnp.float32),
         jnp.dot(wd, t2, preferred_element_type=jnp.float32)], axis=0)
    yscr[0:4, :] = yscr[0:4, :] + y
    t1scr[...] = t1
    t2scr[...] = t2
    s = sd_ref[0, 0]
    d = sd_ref[0, 1]
    sl = s - pid * _D

    @pl.when((sl >= 0) & (sl < _D))
    def _():
        yscr[4:5, :] = t1scr[pl.ds(sl, 1), :]
        yscr[6:7, :] = t2scr[pl.ds(sl, 1), :]

    dl = d - pid * _D

    @pl.when((dl >= 0) & (dl < _D))
    def _():
        yscr[5:6, :] = t1scr[pl.ds(dl, 1), :]
        yscr[7:8, :] = t2scr[pl.ds(dl, 1), :]

    @pl.when(pid == nb - 1)
    def _():
        out_ref[...] = yscr[...]


def kernel(x, edge_index, src, dst, W1a, b1a, W1b, b1b, W2a, b2a, W2b, b2b):
    N, D = x.shape
    E = edge_index.shape[1]
    assert D == _D
    NP = ((N + 2047) // 2048) * 2048
    NB = NP // _D
    info = plsc.get_sparse_core_info()
    NC, NS = info.num_cores, info.num_subcores
    assert NC == 2 and E % (NC * NS) == 0 and NP <= 16384

    CH = 2000 if E % (NC * NS * 2000) == 0 else E // (NC * NS)
    SH1 = E // (NC * NS)
    RB = (E // NS + _D - 1) // _D

    src = jnp.asarray(src, jnp.int32)
    dst = jnp.asarray(dst, jnp.int32)
    sd32 = jnp.concatenate([jnp.full((_L,), src, jnp.int32),
                            jnp.full((_L,), dst, jnp.int32)])
    sd_tc = jnp.stack([src, dst]).reshape(1, 2)

    mesh = plsc.VectorSubcoreMesh(core_axis_name="c", subcore_axis_name="s")
    sc_params = pltpu.CompilerParams(needs_layout_passes=False)

    hist = pl.kernel(
        functools.partial(_sc_hist_body, NP, SH1, CH, NS, E),
        out_type=jax.ShapeDtypeStruct((NC * NS * 3 * NP,), jnp.float32),
        mesh=mesh,
        compiler_params=sc_params,
        scratch_types=[
            pltpu.VMEM((2 * _L,), jnp.int32),
            pltpu.VMEM((CH,), jnp.int32),
            pltpu.VMEM((CH,), jnp.int32),
            pltpu.VMEM((NP,), jnp.float32),
            pltpu.VMEM((NP,), jnp.float32),
            pltpu.VMEM((NP,), jnp.float32),
        ],
    )(edge_index.reshape(-1), sd32)

    xp = jnp.zeros((NP, D), jnp.float32).at[:N].set(x)
    g, dis8, mdis8, ws8, wd8 = pl.pallas_call(
        _tc_prep_body,
        grid=(NB,),
        in_specs=[
            pl.BlockSpec((_D, D), lambda b: (b, 0)),
            pl.BlockSpec((NC * NS, 3, _D), lambda b: (0, 0, b)),
            pl.BlockSpec((D, D), lambda b: (0, 0)),
            pl.BlockSpec((D, D), lambda b: (0, 0)),
            pl.BlockSpec(memory_space=pltpu.SMEM),
        ],
        out_specs=[
            pl.BlockSpec((2, _D, D), lambda b: (0, b, 0)),
            pl.BlockSpec((1, 1, _D), lambda b: (b, 0, 0)),
            pl.BlockSpec((1, 1, _D), lambda b: (b, 0, 0)),
            pl.BlockSpec((1, 1, _D), lambda b: (b, 0, 0)),
            pl.BlockSpec((1, 1, _D), lambda b: (b, 0, 0)),
        ],
        out_shape=[
            jax.ShapeDtypeStruct((2, NP, D), jnp.float32),
            jax.ShapeDtypeStruct((NB, 1, _D), jnp.float32),
            jax.ShapeDtypeStruct((NB, 1, _D), jnp.float32),
            jax.ShapeDtypeStruct((NB, 1, _D), jnp.float32),
            jax.ShapeDtypeStruct((NB, 1, _D), jnp.float32),
        ],
    )(xp, hist.reshape(NC * NS, 3, NP), W1a, W2a, sd_tc)

    mdisf = jnp.concatenate([mdis8.reshape(-1), jnp.zeros((_L,), jnp.float32)])
    disf = dis8.reshape(-1)
    gflat = g.reshape(2 * NP, D)
    zz = jnp.zeros((_D, D), jnp.float32)
    RBP = (E // NS + _D - 1) // _D

    accs = pl.kernel(
        functools.partial(_sc_msg_body, NP, E, RBP, CH, NS),
        out_type=jax.ShapeDtypeStruct((2, NP, D), jnp.float32),
        mesh=mesh,
        compiler_params=sc_params,
        scratch_types=[
            pltpu.VMEM((NP + _L,), jnp.float32),
            pltpu.VMEM((CH,), jnp.int32),
            pltpu.VMEM((CH,), jnp.int32),
            pltpu.VMEM((RBP, _D), jnp.int32),
            pltpu.VMEM((64,), jnp.int32),
            pltpu.VMEM((1, 64), jnp.int32),
            pltpu.VMEM((64, _D), jnp.float32),
            pltpu.SemaphoreType.DMA,
            pltpu.VMEM_SHARED((NP + _D, D), jnp.float32),
        ],
    )(edge_index.reshape(-1), mdisf, gflat, zz)

    res = pl.pallas_call(
        _tc_epi_body,
        grid=(NB,),
        in_specs=[
            pl.BlockSpec((2, _D, D), lambda b: (0, b, 0)),
            pl.BlockSpec((2, _D, D), lambda b: (0, b, 0)),
            pl.BlockSpec((1, 1, _D), lambda b: (b, 0, 0)),
            pl.BlockSpec((1, 1, _D), lambda b: (b, 0, 0)),
            pl.BlockSpec((1, 1, _D), lambda b: (b, 0, 0)),
            pl.BlockSpec((1, D), lambda b: (0, 0)),
            pl.BlockSpec((1, D), lambda b: (0, 0)),
            pl.BlockSpec((D, D), lambda b: (0, 0)),
            pl.BlockSpec((D, D), lambda b: (0, 0)),
            pl.BlockSpec(memory_space=pltpu.SMEM),
        ],
        out_specs=pl.BlockSpec((8, _D), lambda b: (0, 0)),
        out_shape=jax.ShapeDtypeStruct((8, _D), jnp.float32),
        scratch_shapes=[
            pltpu.VMEM((8, _D), jnp.float32),
            pltpu.VMEM((_D, D), jnp.float32),
            pltpu.VMEM((_D, D), jnp.float32),
        ],
    )(accs, g, dis8, ws8, wd8, b1a.reshape(1, D), b2a.reshape(1, D),
      W1b, W2b, sd_tc)

    dis_s = disf[src]
    dis_d = disf[dst]
    y1s = dis_s * res[0] + dis_s * dis_s * res[4] + b1b
    y1d = dis_d * res[1] + dis_d * dis_d * res[5] + b1b
    y2s = dis_s * res[2] + dis_s * dis_s * res[6] + b2b
    y2d = dis_d * res[3] + dis_d * dis_d * res[7] + b2b
    l1 = jnp.sum(y1s * y1d)
    l2 = jnp.sum(y2s * y2d)
    return jnp.stack([l1, l2])


# speedup vs baseline: 54.5887x; 1.0956x over previous
import functools

import jax
import jax.numpy as jnp
from jax import lax
from jax.experimental import pallas as pl
from jax.experimental.pallas import tpu as pltpu
from jax.experimental.pallas import tpu_sc as plsc

_L = 16
_D = 128


def _sc_hist_body(NP, SH, CH, NS, E, edge_ref, sd_ref, out_ref,
                  sd_v, srcc_v, dstc_v, deg_v, cs_v, cd_v):
    c = lax.axis_index("c")
    sid = lax.axis_index("s")
    wid = c * NS + sid
    z16 = jnp.zeros((_L,), jnp.float32)

    def zero_body(i, _):
        deg_v[pl.ds(i * _L, _L)] = z16
        cs_v[pl.ds(i * _L, _L)] = z16
        cd_v[pl.ds(i * _L, _L)] = z16
        return 0
    lax.fori_loop(0, NP // _L, zero_body, 0)

    pltpu.sync_copy(sd_ref, sd_v)
    s_splat = sd_v[pl.ds(0, _L)]
    d_splat = sd_v[pl.ds(_L, _L)]
    ones16 = jnp.ones((_L,), jnp.float32)

    for ch in range(SH // CH):
        base = wid * SH + ch * CH
        pltpu.sync_copy(edge_ref.at[pl.ds(base, CH)], srcc_v)
        pltpu.sync_copy(edge_ref.at[pl.ds(E + base, CH)], dstc_v)

        def body(k, _):
            for u in range(5):
                i = k * 5 + u
                sv = srcc_v[pl.ds(i * _L, _L)]
                dv = dstc_v[pl.ds(i * _L, _L)]
                plsc.addupdate_scatter(deg_v, [dv], ones16, mask=dv >= 0)
                plsc.addupdate_scatter(cs_v, [sv], ones16, mask=dv == s_splat)
                plsc.addupdate_scatter(cd_v, [sv], ones16, mask=dv == d_splat)
            return 0
        lax.fori_loop(0, CH // _L // 5, body, 0)

    pltpu.sync_copy(deg_v, out_ref.at[pl.ds((wid * 3 + 0) * NP, NP)])
    pltpu.sync_copy(cs_v, out_ref.at[pl.ds((wid * 3 + 1) * NP, NP)])
    pltpu.sync_copy(cd_v, out_ref.at[pl.ds((wid * 3 + 2) * NP, NP)])


def _sc_msg_body(NP, E, RBP, CH, NS, edge_ref, mdis_ref, g_ref, zz_ref,
                 out_ref, mdis_v, srcc_v, dstc_v, packed, src_stage, dst_stage,
                 rowbuf, sem, acc):
    c = lax.axis_index("c")
    sid = lax.axis_index("s")
    SH = E // NS
    ACCR = NP + _D
    G = 64
    iota16 = lax.iota(jnp.int32, _L)

    pltpu.sync_copy(mdis_ref, mdis_v)

    head_base = c * NP
    fill = jnp.full((_L,), NP * 16384, jnp.int32)

    def prefill(i, _):
        p = i * _L + iota16
        ri = jnp.right_shift(p, 7)
        ci = jnp.bitwise_and(p, _D - 1)
        plsc.store_scatter(packed, [ri, ci], fill)
        return 0
    lax.fori_loop(0, RBP * _D // _L, prefill, 0)

    rows_pt = ACCR // NS
    done = 0
    for chunk in [_D] * (rows_pt // _D) + ([rows_pt % _D] if rows_pt % _D else []):
        pltpu.sync_copy(zz_ref.at[pl.ds(0, chunk)],
                        acc.at[pl.ds(sid * rows_pt + done, chunk)])
        done += chunk
    plsc.subcore_barrier()

    off = jnp.int32(0)
    UN = 5
    for ch in range(SH // CH):
        base = sid * SH + ch * CH
        pltpu.sync_copy(edge_ref.at[pl.ds(base, CH)], srcc_v)
        pltpu.sync_copy(edge_ref.at[pl.ds(E + base, CH)], dstc_v)

        def body(k, off):
            for u in range(UN):
                i = k * UN + u
                sv = srcc_v[pl.ds(i * _L, _L)]
                dv = dstc_v[pl.ds(i * _L, _L)]
                mk = plsc.load_gather(mdis_v, [dv])
                m = mk > 0.0
                mi = m.astype(jnp.int32)
                cs = plsc.cumsum(mi)
                pos = off + cs - mi
                ri = jnp.right_shift(pos, 7)
                ci = jnp.bitwise_and(pos, _D - 1)
                plsc.store_scatter(packed, [ri, ci], dv * 16384 + sv, mask=m)
                off = off + cs[_L - 1]
            return off
        off = lax.fori_loop(0, CH // _L // UN, body, off)

    zero16 = jnp.zeros((_L,), jnp.int32)
    nch = jnp.right_shift(off + (G - 1), 6)

    def chunk_body(i, _):
        i_splat = jnp.full((_L,), jnp.right_shift(i, 1), jnp.int32)
        lane0 = jnp.bitwise_and(i, 1) * G

        def upk(j, _):
            lanes = lane0 + j * _L + iota16
            pv = plsc.load_gather(packed, [i_splat, lanes])
            dv = jnp.right_shift(pv, 14)
            sv = jnp.bitwise_and(pv, 16383)
            src_stage[pl.ds(j * _L, _L)] = sv + head_base
            plsc.store_scatter(dst_stage, [zero16, j * _L + iota16], dv)
            return 0
        lax.fori_loop(0, G // _L, upk, 0)
        pltpu.async_copy(g_ref.at[src_stage], rowbuf, sem).wait()
        pltpu.sync_copy(rowbuf, acc.at[dst_stage.at[0]], add=True)
        return 0
    lax.fori_loop(0, nch, chunk_body, 0)

    plsc.subcore_barrier()

    rows_wb = NP // NS
    done = 0
    for _unused in range(rows_wb // _D):
        pltpu.sync_copy(acc.at[pl.ds(sid * rows_wb + done, _D)],
                        out_ref.at[c, pl.ds(sid * rows_wb + done, _D)])
        done += _D


def _tc_prep_body(x_ref, hist_ref, w1_ref, w2_ref, sd_ref,
                  g_ref, dis_ref, mark_ref, ws_ref, wd_ref):
    pid = pl.program_id(0)
    h = jnp.sum(hist_ref[...], axis=0)
    deg = h[0:1, :] + 1.0
    cs = h[1:2, :]
    cd = h[2:3, :]
    dis = 1.0 / jnp.sqrt(deg)
    ids = pid * _D + lax.broadcasted_iota(jnp.int32, (1, _D), 1)
    s = sd_ref[0, 0]
    d = sd_ref[0, 1]
    marked = (cs + cd > 0.0) | (ids == s) | (ids == d)
    dis_ref[0] = dis
    mark_ref[0] = jnp.where(marked, dis, -dis)
    ws_ref[0] = cs * dis
    wd_ref[0] = cd * dis
    ey = (lax.broadcasted_iota(jnp.int32, (_D, _D), 0)
          == lax.broadcasted_iota(jnp.int32, (_D, _D), 1)).astype(jnp.float32)
    dis_col = lax.dot_general(ey, dis, (((1,), (1,)), ((), ())),
                              preferred_element_type=jnp.float32)
    xb = x_ref[...]
    g_ref[0] = jnp.dot(xb, w1_ref[...], preferred_element_type=jnp.float32) * dis_col
    g_ref[1] = jnp.dot(xb, w2_ref[...], preferred_element_type=jnp.float32) * dis_col


def _tc_epi_body(acc_ref, g_ref, dis_ref, ws_ref, wd_ref, b1a_ref, b2a_ref,
                 w1b_ref, w2b_ref, sd_ref, out_ref, yscr, t1scr, t2scr):
    pid = pl.program_id(0)
    nb = pl.num_programs(0)

    @pl.when(pid == 0)
    def _():
        yscr[...] = jnp.zeros((8, _D), jnp.float32)

    dis = dis_ref[0]
    ey = (lax.broadcasted_iota(jnp.int32, (_D, _D), 0)
          == lax.broadcasted_iota(jnp.int32, (_D, _D), 1)).astype(jnp.float32)
    dis_col = lax.dot_general(ey, dis, (((1,), (1,)), ((), ())),
                              preferred_element_type=jnp.float32)
    z1 = (acc_ref[0] + g_ref[0]) * dis_col + b1a_ref[...]
    z2 = (acc_ref[1] + g_ref[1]) * dis_col + b2a_ref[...]
    t1 = jnp.dot(jax.nn.relu(z1), w1b_ref[...], preferred_element_type=jnp.float32)
    t2 = jnp.dot(jax.nn.relu(z2), w2b_ref[...], preferred_element_type=jnp.float32)
    ws = ws_ref[0]
    wd = wd_ref[0]
    y = jnp.concatenate(
        [jnp.dot(ws, t1, preferred_element_type=jnp.float32),
         jnp.dot(wd, t1, preferred_element_type=jnp.float32),
         jnp.dot(ws, t2, preferred_element_type=jnp.float32),
         jnp.dot(wd, t2, preferred_element_type=jnp.float32)], axis=0)
    yscr[0:4, :] = yscr[0:4, :] + y
    t1scr[...] = t1
    t2scr[...] = t2
    s = sd_ref[0, 0]
    d = sd_ref[0, 1]
    sl = s - pid * _D

    @pl.when((sl >= 0) & (sl < _D))
    def _():
        yscr[4:5, :] = t1scr[pl.ds(sl, 1), :]
        yscr[6:7, :] = t2scr[pl.ds(sl, 1), :]

    dl = d - pid * _D

    @pl.when((dl >= 0) & (dl < _D))
    def _():
        yscr[5:6, :] = t1scr[pl.ds(dl, 1), :]
        yscr[7:8, :] = t2scr[pl.ds(dl, 1), :]

    @pl.when(pid == nb - 1)
    def _():
        out_ref[...] = yscr[...]


def kernel(x, edge_index, src, dst, W1a, b1a, W1b, b1b, W2a, b2a, W2b, b2b):
    N, D = x.shape
    E = edge_index.shape[1]
    assert D == _D
    NP = ((N + 2047) // 2048) * 2048
    NB = NP // _D
    info = plsc.get_sparse_core_info()
    NC, NS = info.num_cores, info.num_subcores
    assert NC == 2 and E % (NC * NS) == 0 and NP <= 16384

    CH = 2000 if E % (NC * NS * 2000) == 0 else E // (NC * NS)
    SH1 = E // (NC * NS)
    RB = (E // NS + _D - 1) // _D

    src = jnp.asarray(src, jnp.int32)
    dst = jnp.asarray(dst, jnp.int32)
    sd32 = jnp.concatenate([jnp.full((_L,), src, jnp.int32),
                            jnp.full((_L,), dst, jnp.int32)])
    sd_tc = jnp.stack([src, dst]).reshape(1, 2)

    mesh = plsc.VectorSubcoreMesh(core_axis_name="c", subcore_axis_name="s")
    sc_params = pltpu.CompilerParams(needs_layout_passes=False)

    hist = pl.kernel(
        functools.partial(_sc_hist_body, NP, SH1, CH, NS, E),
        out_type=jax.ShapeDtypeStruct((NC * NS * 3 * NP,), jnp.float32),
        mesh=mesh,
        compiler_params=sc_params,
        scratch_types=[
            pltpu.VMEM((2 * _L,), jnp.int32),
            pltpu.VMEM((CH,), jnp.int32),
            pltpu.VMEM((CH,), jnp.int32),
            pltpu.VMEM((NP,), jnp.float32),
            pltpu.VMEM((NP,), jnp.float32),
            pltpu.VMEM((NP,), jnp.float32),
        ],
    )(edge_index.reshape(-1), sd32)

    xp = jnp.zeros((NP, D), jnp.float32).at[:N].set(x)
    g, dis8, mdis8, ws8, wd8 = pl.pallas_call(
        _tc_prep_body,
        grid=(NB,),
        in_specs=[
            pl.BlockSpec((_D, D), lambda b: (b, 0)),
            pl.BlockSpec((NC * NS, 3, _D), lambda b: (0, 0, b)),
            pl.BlockSpec((D, D), lambda b: (0, 0)),
            pl.BlockSpec((D, D), lambda b: (0, 0)),
            pl.BlockSpec(memory_space=pltpu.SMEM),
        ],
        out_specs=[
            pl.BlockSpec((2, _D, D), lambda b: (0, b, 0)),
            pl.BlockSpec((1, 1, _D), lambda b: (b, 0, 0)),
            pl.BlockSpec((1, 1, _D), lambda b: (b, 0, 0)),
            pl.BlockSpec((1, 1, _D), lambda b: (b, 0, 0)),
            pl.BlockSpec((1, 1, _D), lambda b: (b, 0, 0)),
        ],
        out_shape=[
            jax.ShapeDtypeStruct((2, NP, D), jnp.float32),
            jax.ShapeDtypeStruct((NB, 1, _D), jnp.float32),
            jax.ShapeDtypeStruct((NB, 1, _D), jnp.float32),
            jax.ShapeDtypeStruct((NB, 1, _D), jnp.float32),
            jax.ShapeDtypeStruct((NB, 1, _D), jnp.float32),
        ],
    )(xp, hist.reshape(NC * NS, 3, NP), W1a, W2a, sd_tc)

    mdisf = jnp.concatenate([mdis8.reshape(-1), jnp.zeros((_L,), jnp.float32)])
    disf = dis8.reshape(-1)
    gflat = g.reshape(2 * NP, D)
    zz = jnp.zeros((_D, D), jnp.float32)
    RBP = (E // NS + _D - 1) // _D

    accs = pl.kernel(
        functools.partial(_sc_msg_body, NP, E, RBP, CH, NS),
        out_type=jax.ShapeDtypeStruct((2, NP, D), jnp.float32),
        mesh=mesh,
        compiler_params=sc_params,
        scratch_types=[
            pltpu.VMEM((NP + _L,), jnp.float32),
            pltpu.VMEM((CH,), jnp.int32),
            pltpu.VMEM((CH,), jnp.int32),
            pltpu.VMEM((RBP, _D), jnp.int32),
            pltpu.VMEM((64,), jnp.int32),
            pltpu.VMEM((1, 64), jnp.int32),
            pltpu.VMEM((64, _D), jnp.float32),
            pltpu.SemaphoreType.DMA,
            pltpu.VMEM_SHARED((NP + _D, D), jnp.float32),
        ],
    )(edge_index.reshape(-1), mdisf, gflat, zz)

    res = pl.pallas_call(
        _tc_epi_body,
        grid=(NB,),
        in_specs=[
            pl.BlockSpec((2, _D, D), lambda b: (0, b, 0)),
            pl.BlockSpec((2, _D, D), lambda b: (0, b, 0)),
            pl.BlockSpec((1, 1, _D), lambda b: (b, 0, 0)),
            pl.BlockSpec((1, 1, _D), lambda b: (b, 0, 0)),
            pl.BlockSpec((1, 1, _D), lambda b: (b, 0, 0)),
            pl.BlockSpec((1, D), lambda b: (0, 0)),
            pl.BlockSpec((1, D), lambda b: (0, 0)),
            pl.BlockSpec((D, D), lambda b: (0, 0)),
            pl.BlockSpec((D, D), lambda b: (0, 0)),
            pl.BlockSpec(memory_space=pltpu.SMEM),
        ],
        out_specs=pl.BlockSpec((8, _D), lambda b: (0, 0)),
        out_shape=jax.ShapeDtypeStruct((8, _D), jnp.float32),
        scratch_shapes=[
            pltpu.VMEM((8, _D), jnp.float32),
            pltpu.VMEM((_D, D), jnp.float32),
            pltpu.VMEM((_D, D), jnp.float32),
        ],
    )(accs, g, dis8, ws8, wd8, b1a.reshape(1, D), b2a.reshape(1, D),
      W1b, W2b, sd_tc)

    dis_s = disf[src]
    dis_d = disf[dst]
    y1s = dis_s * res[0] + dis_s * dis_s * res[4] + b1b
    y1d = dis_d * res[1] + dis_d * dis_d * res[5] + b1b
    y2s = dis_s * res[2] + dis_s * dis_s * res[6] + b2b
    y2d = dis_d * res[3] + dis_d * dis_d * res[7] + b2b
    l1 = jnp.sum(y1s * y1d)
    l2 = jnp.sum(y2s * y2d)
    return jnp.stack([l1, l2])


# speedup vs baseline: 91.4168x; 1.6746x over previous
import functools

import jax
import jax.numpy as jnp
from jax import lax
from jax.experimental import pallas as pl
from jax.experimental.pallas import tpu as pltpu
from jax.experimental.pallas import tpu_sc as plsc

_L = 16
_D = 128


def _sc_hist_body(NP, SH, CH, NS, NC, E, edge_ref, sd_ref, out_ref,
                  sd_v, srcc_v, dstc_v, deg_v, cs_v, cd_v):
    c = lax.axis_index("c")
    sid = lax.axis_index("s")
    wid = c * NS + sid
    z16 = jnp.zeros((_L,), jnp.float32)

    def zero_body(i, _):
        deg_v[pl.ds(i * _L, _L)] = z16
        cs_v[pl.ds(i * _L, _L)] = z16
        cd_v[pl.ds(i * _L, _L)] = z16
        return 0
    lax.fori_loop(0, NP // _L, zero_body, 0)

    pltpu.sync_copy(sd_ref, sd_v)
    s_splat = sd_v[pl.ds(0, _L)]
    d_splat = sd_v[pl.ds(_L, _L)]
    ones16 = jnp.ones((_L,), jnp.float32)

    for ch in range(SH // CH):
        base = wid * SH + ch * CH
        pltpu.sync_copy(edge_ref.at[pl.ds(base, CH)], srcc_v)
        pltpu.sync_copy(edge_ref.at[pl.ds(E + base, CH)], dstc_v)

        def body(k, _):
            for u in range(5):
                i = k * 5 + u
                sv = srcc_v[pl.ds(i * _L, _L)]
                dv = dstc_v[pl.ds(i * _L, _L)]
                plsc.addupdate_scatter(deg_v, [dv], ones16, mask=dv >= 0)
                plsc.addupdate_scatter(cs_v, [sv], ones16, mask=dv == s_splat)
                plsc.addupdate_scatter(cd_v, [sv], ones16, mask=dv == d_splat)
            return 0
        lax.fori_loop(0, CH // _L // 5, body, 0)

    nw = NC * NS
    pltpu.sync_copy(deg_v, out_ref.at[pl.ds((0 * nw + wid) * NP, NP)])
    pltpu.sync_copy(cs_v, out_ref.at[pl.ds((1 * nw + wid) * NP, NP)])
    pltpu.sync_copy(cd_v, out_ref.at[pl.ds((2 * nw + wid) * NP, NP)])


def _sc_msg_body(NP, E, RBP, CH, NS, edge_ref, mdis_ref, g_ref, zz_ref,
                 out_ref, mdis_v, srcc_v, dstc_v, packed, src_stage, dst_stage,
                 rowbuf, sem, acc):
    c = lax.axis_index("c")
    sid = lax.axis_index("s")
    SH = E // NS
    ACCR = NP + _D
    G = 64
    iota16 = lax.iota(jnp.int32, _L)

    pltpu.sync_copy(mdis_ref, mdis_v)

    head_base = c * NP

    rows_pt = ACCR // NS
    done = 0
    for chunk in [_D] * (rows_pt // _D) + ([rows_pt % _D] if rows_pt % _D else []):
        pltpu.sync_copy(zz_ref.at[pl.ds(0, chunk)],
                        acc.at[pl.ds(sid * rows_pt + done, chunk)])
        done += chunk
    plsc.subcore_barrier()

    off = jnp.int32(0)
    UN = 5
    for ch in range(SH // CH):
        base = sid * SH + ch * CH
        pltpu.sync_copy(edge_ref.at[pl.ds(base, CH)], srcc_v)
        pltpu.sync_copy(edge_ref.at[pl.ds(E + base, CH)], dstc_v)

        def body(k, off):
            for u in range(UN):
                i = k * UN + u
                sv = srcc_v[pl.ds(i * _L, _L)]
                dv = dstc_v[pl.ds(i * _L, _L)]
                mk = plsc.load_gather(mdis_v, [dv])
                m = mk > 0.0
                plsc.store_compressed(packed.at[pl.ds(off, _L)],
                                      dv * 16384 + sv, mask=m)
                off = off + plsc.all_reduce_population_count(m)[0]
            return off
        off = lax.fori_loop(0, CH // _L // UN, body, off)

    zero16 = jnp.zeros((_L,), jnp.int32)
    nch = jnp.right_shift(off + (G - 1), 6)

    def chunk_body(i, _):
        def upk(j, _):
            pos = i * G + j * _L + iota16
            valid = pos < off
            pv = plsc.load_gather(packed, [pos])
            dv = jnp.where(valid, jnp.right_shift(pv, 14), NP)
            sv = jnp.where(valid, jnp.bitwise_and(pv, 16383), 0)
            src_stage[pl.ds(j * _L, _L)] = sv + head_base
            plsc.store_scatter(dst_stage, [zero16, j * _L + iota16], dv)
            return 0
        lax.fori_loop(0, G // _L, upk, 0)
        pltpu.async_copy(g_ref.at[src_stage], rowbuf, sem).wait()
        pltpu.sync_copy(rowbuf, acc.at[dst_stage.at[0]], add=True)
        return 0
    lax.fori_loop(0, nch, chunk_body, 0)

    plsc.subcore_barrier()

    rows_wb = NP // NS
    done = 0
    for _unused in range(rows_wb // _D):
        pltpu.sync_copy(acc.at[pl.ds(sid * rows_wb + done, _D)],
                        out_ref.at[c, pl.ds(sid * rows_wb + done, _D)])
        done += _D


def _col(row_mat, ey):
    parts = [lax.dot_general(ey, row_mat[t:t + 1, :], (((1,), (1,)), ((), ())),
                             preferred_element_type=jnp.float32)
             for t in range(row_mat.shape[0])]
    return jnp.concatenate(parts, axis=0)


def _tc_prep_body(BR, x_ref, hist_ref, w1_ref, w2_ref, sd_ref,
                  g_ref, dis_ref, mdis_ref, ws_ref, wd_ref):
    S = BR // _D
    pid = pl.program_id(0)
    h = hist_ref[:, 0]
    nw = h.shape[0] // 3
    deg = jnp.sum(h[0:nw], axis=0) + 1.0
    cs = jnp.sum(h[nw:2 * nw], axis=0)
    cd = jnp.sum(h[2 * nw:], axis=0)
    dis = 1.0 / jnp.sqrt(deg)
    ids = (pid * BR + lax.broadcasted_iota(jnp.int32, (S, _D), 0) * _D
           + lax.broadcasted_iota(jnp.int32, (S, _D), 1))
    s = sd_ref[0, 0]
    d = sd_ref[0, 1]
    marked = (cs + cd > 0.0) | (ids == s) | (ids == d)
    dis_ref[0] = dis
    mdis_ref[0] = jnp.where(marked, dis, -dis)
    ws_ref[0] = cs * dis
    wd_ref[0] = cd * dis
    ey = (lax.broadcasted_iota(jnp.int32, (_D, _D), 0)
          == lax.broadcasted_iota(jnp.int32, (_D, _D), 1)).astype(jnp.float32)
    dis_col = _col(dis, ey)
    xb = x_ref[...]
    g_ref[0] = jnp.dot(xb, w1_ref[...], preferred_element_type=jnp.float32) * dis_col
    g_ref[1] = jnp.dot(xb, w2_ref[...], preferred_element_type=jnp.float32) * dis_col


def _tc_epi_body(BR, acc_ref, g_ref, dis_ref, ws_ref, wd_ref, b1a_ref, b2a_ref,
                 w1b_ref, w2b_ref, sd_ref, out_ref, yscr, t1scr, t2scr):
    pid = pl.program_id(0)
    nb = pl.num_programs(0)

    @pl.when(pid == 0)
    def _():
        yscr[...] = jnp.zeros((8, _D), jnp.float32)

    ey = (lax.broadcasted_iota(jnp.int32, (_D, _D), 0)
          == lax.broadcasted_iota(jnp.int32, (_D, _D), 1)).astype(jnp.float32)
    dis_col = _col(dis_ref[0], ey)
    ws_col = _col(ws_ref[0], ey)
    wd_col = _col(wd_ref[0], ey)
    z1 = (acc_ref[0] + g_ref[0]) * dis_col + b1a_ref[...]
    z2 = (acc_ref[1] + g_ref[1]) * dis_col + b2a_ref[...]
    t1 = jnp.dot(jax.nn.relu(z1), w1b_ref[...], preferred_element_type=jnp.float32)
    t2 = jnp.dot(jax.nn.relu(z2), w2b_ref[...], preferred_element_type=jnp.float32)
    cdim = (((0,), (0,)), ((), ()))
    y = jnp.concatenate(
        [lax.dot_general(ws_col, t1, cdim, preferred_element_type=jnp.float32),
         lax.dot_general(wd_col, t1, cdim, preferred_element_type=jnp.float32),
         lax.dot_general(ws_col, t2, cdim, preferred_element_type=jnp.float32),
         lax.dot_general(wd_col, t2, cdim, preferred_element_type=jnp.float32)],
        axis=0)
    yscr[0:4, :] = yscr[0:4, :] + y
    t1scr[...] = t1
    t2scr[...] = t2
    s = sd_ref[0, 0]
    d = sd_ref[0, 1]
    BRl = t1scr.shape[0]
    sl = s - pid * BRl

    @pl.when((sl >= 0) & (sl < BRl))
    def _():
        yscr[4:5, :] = t1scr[pl.ds(sl, 1), :]
        yscr[6:7, :] = t2scr[pl.ds(sl, 1), :]

    dl = d - pid * BRl

    @pl.when((dl >= 0) & (dl < BRl))
    def _():
        yscr[5:6, :] = t1scr[pl.ds(dl, 1), :]
        yscr[7:8, :] = t2scr[pl.ds(dl, 1), :]

    @pl.when(pid == nb - 1)
    def _():
        out_ref[...] = yscr[...]


def kernel(x, edge_index, src, dst, W1a, b1a, W1b, b1b, W2a, b2a, W2b, b2b):
    N, D = x.shape
    E = edge_index.shape[1]
    assert D == _D
    NP = ((N + 2047) // 2048) * 2048
    NB = NP // _D
    info = plsc.get_sparse_core_info()
    NC, NS = info.num_cores, info.num_subcores
    assert NC == 2 and E % (NC * NS) == 0 and NP <= 16384

    CH = 2000 if E % (NC * NS * 2000) == 0 else E // (NC * NS)
    SH1 = E // (NC * NS)
    RB = (E // NS + _D - 1) // _D

    src = jnp.asarray(src, jnp.int32)
    dst = jnp.asarray(dst, jnp.int32)
    sd32 = jnp.concatenate([jnp.full((_L,), src, jnp.int32),
                            jnp.full((_L,), dst, jnp.int32)])
    sd_tc = jnp.stack([src, dst]).reshape(1, 2)

    mesh = plsc.VectorSubcoreMesh(core_axis_name="c", subcore_axis_name="s")
    sc_params = pltpu.CompilerParams(needs_layout_passes=False)

    hist = pl.kernel(
        functools.partial(_sc_hist_body, NP, SH1, CH, NS, NC, E),
        out_type=jax.ShapeDtypeStruct((NC * NS * 3 * NP,), jnp.float32),
        mesh=mesh,
        compiler_params=sc_params,
        scratch_types=[
            pltpu.VMEM((2 * _L,), jnp.int32),
            pltpu.VMEM((CH,), jnp.int32),
            pltpu.VMEM((CH,), jnp.int32),
            pltpu.VMEM((NP,), jnp.float32),
            pltpu.VMEM((NP,), jnp.float32),
            pltpu.VMEM((NP,), jnp.float32),
        ],
    )(edge_index.reshape(-1), sd32)

    BR = 1024
    NBB = NP // BR
    S = BR // _D
    xp = jnp.zeros((NP, D), jnp.float32).at[:N].set(x)
    hist4 = hist.reshape(3 * NC * NS, NBB, S, _D)
    g, dis8, mdis8, ws8, wd8 = pl.pallas_call(
        functools.partial(_tc_prep_body, BR),
        grid=(NBB,),
        in_specs=[
            pl.BlockSpec((BR, D), lambda b: (b, 0)),
            pl.BlockSpec((3 * NC * NS, 1, S, _D), lambda b: (0, b, 0, 0)),
            pl.BlockSpec((D, D), lambda b: (0, 0)),
            pl.BlockSpec((D, D), lambda b: (0, 0)),
            pl.BlockSpec(memory_space=pltpu.SMEM),
        ],
        out_specs=[
            pl.BlockSpec((2, BR, D), lambda b: (0, b, 0)),
            pl.BlockSpec((1, S, _D), lambda b: (b, 0, 0)),
            pl.BlockSpec((1, S, _D), lambda b: (b, 0, 0)),
            pl.BlockSpec((1, S, _D), lambda b: (b, 0, 0)),
            pl.BlockSpec((1, S, _D), lambda b: (b, 0, 0)),
        ],
        out_shape=[
            jax.ShapeDtypeStruct((2, NP, D), jnp.float32),
            jax.ShapeDtypeStruct((NBB, S, _D), jnp.float32),
            jax.ShapeDtypeStruct((NBB, S, _D), jnp.float32),
            jax.ShapeDtypeStruct((NBB, S, _D), jnp.float32),
            jax.ShapeDtypeStruct((NBB, S, _D), jnp.float32),
        ],
    )(xp, hist4, W1a, W2a, sd_tc)

    mdisf = mdis8.reshape(-1)
    disf = dis8.reshape(-1)
    gflat = g.reshape(2 * NP, D)
    zz = jnp.zeros((_D, D), jnp.float32)
    RBP = (E // NS + _D - 1) // _D

    accs = pl.kernel(
        functools.partial(_sc_msg_body, NP, E, RBP, CH, NS),
        out_type=jax.ShapeDtypeStruct((2, NP, D), jnp.float32),
        mesh=mesh,
        compiler_params=sc_params,
        scratch_types=[
            pltpu.VMEM((NP,), jnp.float32),
            pltpu.VMEM((CH,), jnp.int32),
            pltpu.VMEM((CH,), jnp.int32),
            pltpu.VMEM((RBP * _D,), jnp.int32),
            pltpu.VMEM((64,), jnp.int32),
            pltpu.VMEM((1, 64), jnp.int32),
            pltpu.VMEM((64, _D), jnp.float32),
            pltpu.SemaphoreType.DMA,
            pltpu.VMEM_SHARED((NP + _D, D), jnp.float32),
        ],
    )(edge_index.reshape(-1), mdisf, gflat, zz)

    res = pl.pallas_call(
        functools.partial(_tc_epi_body, BR),
        grid=(NBB,),
        in_specs=[
            pl.BlockSpec((2, BR, D), lambda b: (0, b, 0)),
            pl.BlockSpec((2, BR, D), lambda b: (0, b, 0)),
            pl.BlockSpec((1, S, _D), lambda b: (b, 0, 0)),
            pl.BlockSpec((1, S, _D), lambda b: (b, 0, 0)),
            pl.BlockSpec((1, S, _D), lambda b: (b, 0, 0)),
            pl.BlockSpec((1, D), lambda b: (0, 0)),
            pl.BlockSpec((1, D), lambda b: (0, 0)),
            pl.BlockSpec((D, D), lambda b: (0, 0)),
            pl.BlockSpec((D, D), lambda b: (0, 0)),
            pl.BlockSpec(memory_space=pltpu.SMEM),
        ],
        out_specs=pl.BlockSpec((8, _D), lambda b: (0, 0)),
        out_shape=jax.ShapeDtypeStruct((8, _D), jnp.float32),
        scratch_shapes=[
            pltpu.VMEM((8, _D), jnp.float32),
            pltpu.VMEM((BR, D), jnp.float32),
            pltpu.VMEM((BR, D), jnp.float32),
        ],
    )(accs, g, dis8, ws8, wd8, b1a.reshape(1, D), b2a.reshape(1, D),
      W1b, W2b, sd_tc)

    dis_s = disf[src]
    dis_d = disf[dst]
    y1s = dis_s * res[0] + dis_s * dis_s * res[4] + b1b
    y1d = dis_d * res[1] + dis_d * dis_d * res[5] + b1b
    y2s = dis_s * res[2] + dis_s * dis_s * res[6] + b2b
    y2d = dis_d * res[3] + dis_d * dis_d * res[7] + b2b
    l1 = jnp.sum(y1s * y1d)
    l2 = jnp.sum(y2s * y2d)
    return jnp.stack([l1, l2])
